# baseline (speedup 1.0000x reference)
import functools

import jax
import jax.numpy as jnp
from jax import lax
from jax.experimental import pallas as pl
from jax.experimental.pallas import tpu as pltpu
from jax.experimental.pallas import tpu_sc as plsc

SEQ_LEN = 8192
EMBED_DIM = 1024

_info = plsc.get_sparse_core_info()
_NC, _NS = _info.num_cores, _info.num_subcores
_NW = _NC * _NS
_ROWS_PER_W = SEQ_LEN // _NW
_CH = 32
_NCHUNK = _ROWS_PER_W // _CH
_NBUF = 3

_mesh = plsc.VectorSubcoreMesh(core_axis_name="c", subcore_axis_name="s")


@functools.partial(
    pl.kernel,
    mesh=_mesh,
    out_type=jax.ShapeDtypeStruct((SEQ_LEN, EMBED_DIM), jnp.float32),
    scratch_types=[
        pltpu.VMEM((_NBUF, _CH, EMBED_DIM), jnp.float32),
        pltpu.SemaphoreType.DMA((_NBUF,)),
        pltpu.SemaphoreType.DMA((_NBUF,)),
    ],
)
def _copy_rows(emb_hbm, out_hbm, stage, in_sems, out_sems):
    c = lax.axis_index("c")
    s = lax.axis_index("s")
    wid = s * _NC + c
    base = wid * _ROWS_PER_W

    def in_copy(i):
        return pltpu.make_async_copy(
            emb_hbm.at[pl.ds(base + i * _CH, _CH)],
            stage.at[i % _NBUF],
            in_sems.at[i % _NBUF],
        )

    def out_copy(i):
        return pltpu.make_async_copy(
            stage.at[i % _NBUF],
            out_hbm.at[pl.ds(base + i * _CH, _CH)],
            out_sems.at[i % _NBUF],
        )

    for i in range(_NBUF):
        in_copy(i).start()
    out_waited = set()
    for i in range(_NCHUNK):
        in_copy(i).wait()
        out_copy(i).start()
        if i >= 1 and i + _NBUF - 1 < _NCHUNK:
            out_copy(i - 1).wait()
            out_waited.add(i - 1)
            in_copy(i + _NBUF - 1).start()
    for i in range(_NCHUNK):
        if i not in out_waited:
            out_copy(i).wait()


def kernel(x, embedding):
    del x
    return _copy_rows(embedding)
